# baseline (speedup 1.0000x reference)
import functools

import jax
import jax.numpy as jnp
from jax import lax
from jax.experimental import pallas as pl
from jax.experimental.pallas import tpu as pltpu
from jax.experimental.pallas import tpu_sc as plsc

S = 2048
H = 768
E = 8
INTER = 768
LIMIT = 7.0
ALPHA = 1.702

T = 128
NT = S * 2 // T + E
RPAD = NT * T
NC = 16
CH = S // NC
NW = 32


def _plan_body(x_ref, rw_ref, dest_ref, wts_ref, meta_ref):
    xb = x_ref[...]
    logits = jnp.dot(xb, rw_ref[...], preferred_element_type=jnp.float32)
    m = jnp.max(logits, axis=-1, keepdims=True)
    ex = jnp.exp(logits - m)
    scores = ex / jnp.sum(ex, axis=-1, keepdims=True)
    eidx = lax.broadcasted_iota(jnp.int32, (S, E), 1)
    a1 = jnp.min(jnp.where(logits == m, eidx, E), axis=-1, keepdims=True)
    o1 = jnp.where(eidx == a1, 1.0, 0.0)
    neg = jnp.float32(-jnp.inf)
    logits2 = jnp.where(eidx == a1, neg, logits)
    m2 = jnp.max(logits2, axis=-1, keepdims=True)
    a2 = jnp.min(jnp.where(logits2 == m2, eidx, E), axis=-1, keepdims=True)
    o2 = jnp.where(eidx == a2, 1.0, 0.0)

    w0v = jnp.sum(scores * o1, axis=1).reshape(1, S, 1)
    w1v = jnp.sum(scores * o2, axis=1).reshape(1, S, 1)
    wts_ref[pl.ds(0, 1)] = jnp.broadcast_to(w0v, (1, S, 128))
    wts_ref[pl.ds(1, 1)] = jnp.broadcast_to(w1v, (1, S, 128))

    r128 = lax.broadcasted_iota(jnp.int32, (CH, CH), 0)
    c128 = lax.broadcasted_iota(jnp.int32, (CH, CH), 1)
    Ls = jnp.where(r128 > c128, 1.0, 0.0)
    within1 = []
    within2 = []
    tots1 = []
    tots2 = []
    for c in range(NC):
        o1c = o1[c * CH:(c + 1) * CH]
        o2c = o2[c * CH:(c + 1) * CH]
        within1.append(jnp.dot(Ls, o1c, preferred_element_type=jnp.float32))
        within2.append(jnp.dot(Ls, o2c, preferred_element_type=jnp.float32))
        tots1.append(jnp.sum(o1c, axis=0, keepdims=True))
        tots2.append(jnp.sum(o2c, axis=0, keepdims=True))
    tots1 = jnp.concatenate(tots1, axis=0)
    tots2 = jnp.concatenate(tots2, axis=0)
    r16 = lax.broadcasted_iota(jnp.int32, (NC, NC), 0)
    c16 = lax.broadcasted_iota(jnp.int32, (NC, NC), 1)
    L16 = jnp.where(r16 > c16, 1.0, 0.0)
    base1 = jnp.dot(L16, tots1, preferred_element_type=jnp.float32)
    base2 = jnp.dot(L16, tots2, preferred_element_type=jnp.float32)

    c1tot = jnp.sum(tots1, axis=0, keepdims=True)
    c2tot = jnp.sum(tots2, axis=0, keepdims=True)
    counts = c1tot + c2tot
    ntiles = jnp.floor((counts + (T - 1)) * (1.0 / T))
    r8 = lax.broadcasted_iota(jnp.int32, (E, E), 0)
    c8 = lax.broadcasted_iota(jnp.int32, (E, E), 1)
    U8 = jnp.where(r8 < c8, 1.0, 0.0)
    tile_off = jnp.dot(ntiles, U8, preferred_element_type=jnp.float32)
    seg_base = tile_off * float(T)

    for c in range(NC):
        o1c = o1[c * CH:(c + 1) * CH]
        o2c = o2[c * CH:(c + 1) * CH]
        cnt1 = within1[c] + base1[c:c + 1, :]
        cnt2 = within2[c] + base2[c:c + 1, :]
        d0 = jnp.sum(o1c * (seg_base + cnt1), axis=1)
        d1 = jnp.sum(o2c * (seg_base + c1tot + cnt2), axis=1)
        dest_ref[pl.ds(0, 1), pl.ds(c * CH, CH)] = (
            d0.astype(jnp.int32).reshape(1, CH))
        dest_ref[pl.ds(1, 1), pl.ds(c * CH, CH)] = (
            d1.astype(jnp.int32).reshape(1, CH))

    i40 = lax.broadcasted_iota(jnp.int32, (NT, E), 0).astype(jnp.float32)
    eids = jnp.sum(jnp.where(tile_off <= i40, 1.0, 0.0), axis=1) - 1.0
    meta_ref[pl.ds(0, 1), pl.ds(0, NT)] = eids.astype(jnp.int32).reshape(1, NT)
    total = jnp.sum(ntiles).astype(jnp.int32)
    meta_ref[pl.ds(0, 1), pl.ds(NT, 1)] = total.reshape(1, 1)


@jax.jit
def _plan(xb, rw):
    return pl.pallas_call(
        _plan_body,
        grid=(1,),
        in_specs=[
            pl.BlockSpec((S, H), lambda i: (0, 0)),
            pl.BlockSpec((H, E), lambda i: (0, 0)),
        ],
        out_specs=[
            pl.BlockSpec((8, S), lambda i: (0, 0)),
            pl.BlockSpec((2, S, 128), lambda i: (0, 0, 0)),
            pl.BlockSpec((8, 64), lambda i: (0, 0)),
        ],
        out_shape=[
            jax.ShapeDtypeStruct((8, S), jnp.int32),
            jax.ShapeDtypeStruct((2, S, 128), jnp.float32),
            jax.ShapeDtypeStruct((8, 64), jnp.int32),
        ],
        compiler_params=pltpu.CompilerParams(
            dimension_semantics=("arbitrary",),
        ),
    )(xb, rw)


@functools.cache
def _make_dispatch():
    mesh = plsc.VectorSubcoreMesh(core_axis_name="c", subcore_axis_name="s")

    @jax.jit
    @functools.partial(
        pl.kernel,
        mesh=mesh,
        out_type=[
            jax.ShapeDtypeStruct((RPAD, H), jnp.float32),
            jax.ShapeDtypeStruct((RPAD, 128), jnp.float32),
        ],
        scratch_types=[
            pltpu.VMEM((T,), jnp.int32),
            pltpu.VMEM((T, H), jnp.float32),
            pltpu.VMEM((T, 128), jnp.float32),
            pltpu.SemaphoreType.DMA,
            pltpu.SemaphoreType.DMA,
        ],
    )
    def _dispatch(x_hbm, dest_hbm, w16_hbm, xs_hbm, wr_hbm,
                  idx_v, rows_v, wrow_v, sem, sem2):
        wid = lax.axis_index("s") * 2 + lax.axis_index("c")
        k = wid // 16
        tbase = (wid % 16) * T
        pltpu.sync_copy(dest_hbm.at[k, pl.ds(tbase, T)], idx_v)
        pltpu.sync_copy(x_hbm.at[pl.ds(tbase, T)], rows_v)
        pltpu.sync_copy(w16_hbm.at[k, pl.ds(tbase, T)], wrow_v)
        cp1 = pltpu.async_copy(rows_v, xs_hbm.at[idx_v], sem)
        cp2 = pltpu.async_copy(wrow_v, wr_hbm.at[idx_v], sem2)
        cp1.wait()
        cp2.wait()

    return _dispatch


def _ffn_body(meta_ref, xs_ref, wr_ref, wgu_ref, bgu_ref, wd_ref, bd_ref,
              ys_ref):
    i = pl.program_id(0)

    @pl.when(i < meta_ref[NT])
    def _():
        xb = xs_ref[...].astype(jnp.bfloat16)
        gu = jnp.dot(xb, wgu_ref[0], preferred_element_type=jnp.float32)
        gu = gu + bgu_ref[0]
        gate = jnp.minimum(gu[:, :INTER], LIMIT)
        up = jnp.clip(gu[:, INTER:], -LIMIT, LIMIT)
        glu = gate * jax.nn.sigmoid(gate * ALPHA)
        act = (up + 1.0) * glu
        y = jnp.dot(act.astype(jnp.bfloat16), wd_ref[0],
                    preferred_element_type=jnp.float32)
        ys_ref[...] = (y + bd_ref[0]) * wr_ref[...][:, 0:1]


@jax.jit
def _ffn(meta1d, xs, wr, wgu, bgu, wd, bd):
    grid_spec = pltpu.PrefetchScalarGridSpec(
        num_scalar_prefetch=1,
        grid=(NT,),
        in_specs=[
            pl.BlockSpec((T, H), lambda i, m: (i, 0)),
            pl.BlockSpec((T, 128), lambda i, m: (i, 0)),
            pl.BlockSpec((1, H, 2 * INTER), lambda i, m: (m[i], 0, 0)),
            pl.BlockSpec((1, 1, 2 * INTER), lambda i, m: (m[i], 0, 0)),
            pl.BlockSpec((1, H, INTER), lambda i, m: (m[i], 0, 0)),
            pl.BlockSpec((1, 1, H), lambda i, m: (m[i], 0, 0)),
        ],
        out_specs=pl.BlockSpec((T, H), lambda i, m: (i, 0)),
    )
    return pl.pallas_call(
        _ffn_body,
        grid_spec=grid_spec,
        out_shape=jax.ShapeDtypeStruct((RPAD, H), jnp.float32),
        compiler_params=pltpu.CompilerParams(
            dimension_semantics=("arbitrary",),
        ),
    )(meta1d, xs, wr, wgu, bgu.reshape(E, 1, 2 * INTER), wd,
      bd.reshape(E, 1, H))


_TOK_W = S // NW
_TOK_H = _TOK_W // 2


@functools.cache
def _make_combine():
    mesh = plsc.VectorSubcoreMesh(core_axis_name="c", subcore_axis_name="s")

    @jax.jit
    @functools.partial(
        pl.kernel,
        mesh=mesh,
        out_type=jax.ShapeDtypeStruct((S, H), jnp.float32),
        scratch_types=[
            pltpu.VMEM((_TOK_H,), jnp.int32),
            pltpu.VMEM((_TOK_H,), jnp.int32),
            pltpu.VMEM((_TOK_H, H), jnp.float32),
            pltpu.VMEM((_TOK_H, H), jnp.float32),
            pltpu.SemaphoreType.DMA,
            pltpu.SemaphoreType.DMA,
        ],
    )
    def _combine(ys_hbm, dest_hbm, out_hbm, i0, i1, b0, b1, sem0, sem1):
        wid = lax.axis_index("s") * 2 + lax.axis_index("c")
        for half in range(2):
            base = wid * _TOK_W + half * _TOK_H
            pltpu.sync_copy(dest_hbm.at[0, pl.ds(base, _TOK_H)], i0)
            pltpu.sync_copy(dest_hbm.at[1, pl.ds(base, _TOK_H)], i1)
            cp0 = pltpu.async_copy(ys_hbm.at[i0], b0, sem0)
            cp1 = pltpu.async_copy(ys_hbm.at[i1], b1, sem1)
            cp0.wait()
            cp1.wait()

            def body_j(j, _):
                def body_c(c, _):
                    sl = pl.ds(c * 16, 16)
                    b0[j, sl] = b0[j, sl] + b1[j, sl]
                    return 0

                lax.fori_loop(0, H // 16, body_c, 0)
                return 0

            lax.fori_loop(0, _TOK_H, body_j, 0)
            pltpu.sync_copy(b0, out_hbm.at[pl.ds(base, _TOK_H)])

    return _combine


def kernel(hidden_states, router_w, gate_up_proj, gate_up_proj_bias, down_proj,
           down_proj_bias):
    b, s, h = hidden_states.shape
    x = hidden_states.reshape(s, h)
    xb = x.astype(jnp.bfloat16)
    dest, w16, meta = _plan(xb, router_w.astype(jnp.bfloat16))
    xs, wr = _make_dispatch()(x, dest, w16)
    ys = _ffn(meta[0, :NT + 8], xs, wr,
              gate_up_proj.astype(jnp.bfloat16), gate_up_proj_bias,
              down_proj.astype(jnp.bfloat16), down_proj_bias)
    out = _make_combine()(ys, dest)
    return out.reshape(b, s, h)


# speedup vs baseline: 1.2069x; 1.2069x over previous
import functools

import jax
import jax.numpy as jnp
from jax import lax
from jax.experimental import pallas as pl
from jax.experimental.pallas import tpu as pltpu
from jax.experimental.pallas import tpu_sc as plsc

S = 2048
H = 768
E = 8
INTER = 768
LIMIT = 7.0
ALPHA = 1.702

T = 256
TW = 128
NT = S * 2 // T + E
RPAD = NT * T
NC = 16
CH = S // NC
NW = 32


def _plan_body(x_ref, rw_ref, dest_ref, wts_ref, meta_ref):
    xb = x_ref[...]
    logits = jnp.dot(xb, rw_ref[...], preferred_element_type=jnp.float32)
    m = jnp.max(logits, axis=-1, keepdims=True)
    ex = jnp.exp(logits - m)
    scores = ex / jnp.sum(ex, axis=-1, keepdims=True)
    eidx = lax.broadcasted_iota(jnp.int32, (S, E), 1)
    a1 = jnp.min(jnp.where(logits == m, eidx, E), axis=-1, keepdims=True)
    o1 = jnp.where(eidx == a1, 1.0, 0.0)
    neg = jnp.float32(-jnp.inf)
    logits2 = jnp.where(eidx == a1, neg, logits)
    m2 = jnp.max(logits2, axis=-1, keepdims=True)
    a2 = jnp.min(jnp.where(logits2 == m2, eidx, E), axis=-1, keepdims=True)
    o2 = jnp.where(eidx == a2, 1.0, 0.0)

    w0v = jnp.sum(scores * o1, axis=1).reshape(1, S, 1)
    w1v = jnp.sum(scores * o2, axis=1).reshape(1, S, 1)
    wts_ref[pl.ds(0, 1)] = jnp.broadcast_to(w0v, (1, S, 128))
    wts_ref[pl.ds(1, 1)] = jnp.broadcast_to(w1v, (1, S, 128))

    r128 = lax.broadcasted_iota(jnp.int32, (CH, CH), 0)
    c128 = lax.broadcasted_iota(jnp.int32, (CH, CH), 1)
    Ls = jnp.where(r128 > c128, 1.0, 0.0)
    within1 = []
    within2 = []
    tots1 = []
    tots2 = []
    for c in range(NC):
        o1c = o1[c * CH:(c + 1) * CH]
        o2c = o2[c * CH:(c + 1) * CH]
        within1.append(jnp.dot(Ls, o1c, preferred_element_type=jnp.float32))
        within2.append(jnp.dot(Ls, o2c, preferred_element_type=jnp.float32))
        tots1.append(jnp.sum(o1c, axis=0, keepdims=True))
        tots2.append(jnp.sum(o2c, axis=0, keepdims=True))
    tots1 = jnp.concatenate(tots1, axis=0)
    tots2 = jnp.concatenate(tots2, axis=0)
    r16 = lax.broadcasted_iota(jnp.int32, (NC, NC), 0)
    c16 = lax.broadcasted_iota(jnp.int32, (NC, NC), 1)
    L16 = jnp.where(r16 > c16, 1.0, 0.0)
    base1 = jnp.dot(L16, tots1, preferred_element_type=jnp.float32)
    base2 = jnp.dot(L16, tots2, preferred_element_type=jnp.float32)

    c1tot = jnp.sum(tots1, axis=0, keepdims=True)
    c2tot = jnp.sum(tots2, axis=0, keepdims=True)
    counts = c1tot + c2tot
    ntiles = jnp.floor((counts + (T - 1)) * (1.0 / T))
    r8 = lax.broadcasted_iota(jnp.int32, (E, E), 0)
    c8 = lax.broadcasted_iota(jnp.int32, (E, E), 1)
    U8 = jnp.where(r8 < c8, 1.0, 0.0)
    tile_off = jnp.dot(ntiles, U8, preferred_element_type=jnp.float32)
    seg_base = tile_off * float(T)

    for c in range(NC):
        o1c = o1[c * CH:(c + 1) * CH]
        o2c = o2[c * CH:(c + 1) * CH]
        cnt1 = within1[c] + base1[c:c + 1, :]
        cnt2 = within2[c] + base2[c:c + 1, :]
        d0 = jnp.sum(o1c * (seg_base + cnt1), axis=1)
        d1 = jnp.sum(o2c * (seg_base + c1tot + cnt2), axis=1)
        dest_ref[pl.ds(0, 1), pl.ds(c * CH, CH)] = (
            d0.astype(jnp.int32).reshape(1, CH))
        dest_ref[pl.ds(1, 1), pl.ds(c * CH, CH)] = (
            d1.astype(jnp.int32).reshape(1, CH))

    i40 = lax.broadcasted_iota(jnp.int32, (NT, E), 0).astype(jnp.float32)
    eids = jnp.sum(jnp.where(tile_off <= i40, 1.0, 0.0), axis=1) - 1.0
    meta_ref[pl.ds(0, 1), pl.ds(0, NT)] = eids.astype(jnp.int32).reshape(1, NT)
    total = jnp.sum(ntiles).astype(jnp.int32)
    meta_ref[pl.ds(0, 1), pl.ds(NT, 1)] = total.reshape(1, 1)


@jax.jit
def _plan(xb, rw):
    return pl.pallas_call(
        _plan_body,
        grid=(1,),
        in_specs=[
            pl.BlockSpec((S, H), lambda i: (0, 0)),
            pl.BlockSpec((H, E), lambda i: (0, 0)),
        ],
        out_specs=[
            pl.BlockSpec((8, S), lambda i: (0, 0)),
            pl.BlockSpec((2, S, 128), lambda i: (0, 0, 0)),
            pl.BlockSpec((8, 64), lambda i: (0, 0)),
        ],
        out_shape=[
            jax.ShapeDtypeStruct((8, S), jnp.int32),
            jax.ShapeDtypeStruct((2, S, 128), jnp.float32),
            jax.ShapeDtypeStruct((8, 64), jnp.int32),
        ],
        compiler_params=pltpu.CompilerParams(
            dimension_semantics=("arbitrary",),
        ),
    )(xb, rw)


@functools.cache
def _make_dispatch():
    mesh = plsc.VectorSubcoreMesh(core_axis_name="c", subcore_axis_name="s")

    @jax.jit
    @functools.partial(
        pl.kernel,
        mesh=mesh,
        out_type=[
            jax.ShapeDtypeStruct((RPAD, H), jnp.float32),
            jax.ShapeDtypeStruct((RPAD, 128), jnp.float32),
        ],
        scratch_types=[
            pltpu.VMEM((TW,), jnp.int32),
            pltpu.VMEM((2, TW // 2), jnp.int32),
            pltpu.VMEM((TW // 2, H), jnp.float32),
            pltpu.VMEM((TW // 2, H), jnp.float32),
            pltpu.VMEM((TW, 128), jnp.float32),
            pltpu.SemaphoreType.DMA,
            pltpu.SemaphoreType.DMA,
            pltpu.SemaphoreType.DMA,
            pltpu.SemaphoreType.DMA,
        ],
    )
    def _dispatch(x_hbm, dest_hbm, w16_hbm, xs_hbm, wr_hbm,
                  idx_v, idx2_v, rows_a, rows_b, wrow_v,
                  sem_a, sem_b, sem_w, sem_i):
        wid = lax.axis_index("s") * 2 + lax.axis_index("c")
        k = wid // 16
        tbase = (wid % 16) * TW
        HT = TW // 2
        cpi = pltpu.async_copy(dest_hbm.at[k, pl.ds(tbase, TW)], idx_v, sem_i)
        cpa = pltpu.async_copy(x_hbm.at[pl.ds(tbase, HT)], rows_a, sem_a)
        cpb = pltpu.async_copy(x_hbm.at[pl.ds(tbase + HT, HT)], rows_b, sem_b)
        cpw = pltpu.async_copy(w16_hbm.at[k, pl.ds(tbase, TW)], wrow_v, sem_w)
        cpi.wait()
        pltpu.sync_copy(dest_hbm.at[k, pl.ds(tbase, HT)], idx2_v.at[0])
        pltpu.sync_copy(dest_hbm.at[k, pl.ds(tbase + HT, HT)], idx2_v.at[1])
        cpa.wait()
        sca = pltpu.async_copy(rows_a, xs_hbm.at[idx2_v.at[0]], sem_a)
        cpw.wait()
        scw = pltpu.async_copy(wrow_v, wr_hbm.at[idx_v], sem_w)
        cpb.wait()
        scb = pltpu.async_copy(rows_b, xs_hbm.at[idx2_v.at[1]], sem_b)
        sca.wait()
        scw.wait()
        scb.wait()

    return _dispatch


def _ffn_body(meta_ref, xs_ref, wr_ref, wgu_ref, bgu_ref, wd_ref, bd_ref,
              ys_ref):
    i = pl.program_id(0)

    @pl.when(i < meta_ref[NT])
    def _():
        x = xs_ref[...]
        gu = jnp.dot(x, wgu_ref[0], preferred_element_type=jnp.float32)
        gu = gu + bgu_ref[0]
        gate = jnp.minimum(gu[:, :INTER], LIMIT)
        up = jnp.clip(gu[:, INTER:], -LIMIT, LIMIT)
        glu = gate * jax.nn.sigmoid(gate * ALPHA)
        act = (up + 1.0) * glu
        y = jnp.dot(act, wd_ref[0], preferred_element_type=jnp.float32)
        ys_ref[...] = (y + bd_ref[0]) * wr_ref[...][:, 0:1]


@jax.jit
def _ffn(meta1d, xs, wr, wgu, bgu, wd, bd):
    grid_spec = pltpu.PrefetchScalarGridSpec(
        num_scalar_prefetch=1,
        grid=(NT,),
        in_specs=[
            pl.BlockSpec((T, H), lambda i, m: (i, 0)),
            pl.BlockSpec((T, 128), lambda i, m: (i, 0)),
            pl.BlockSpec((1, H, 2 * INTER), lambda i, m: (m[i], 0, 0)),
            pl.BlockSpec((1, 1, 2 * INTER), lambda i, m: (m[i], 0, 0)),
            pl.BlockSpec((1, H, INTER), lambda i, m: (m[i], 0, 0)),
            pl.BlockSpec((1, 1, H), lambda i, m: (m[i], 0, 0)),
        ],
        out_specs=pl.BlockSpec((T, H), lambda i, m: (i, 0)),
    )
    return pl.pallas_call(
        _ffn_body,
        grid_spec=grid_spec,
        out_shape=jax.ShapeDtypeStruct((RPAD, H), jnp.float32),
        compiler_params=pltpu.CompilerParams(
            dimension_semantics=("parallel",),
        ),
    )(meta1d, xs, wr, wgu, bgu.reshape(E, 1, 2 * INTER), wd,
      bd.reshape(E, 1, H))


_TOK_W = S // NW
_TOK_H = _TOK_W // 2


@functools.cache
def _make_combine():
    mesh = plsc.VectorSubcoreMesh(core_axis_name="c", subcore_axis_name="s")

    @jax.jit
    @functools.partial(
        pl.kernel,
        mesh=mesh,
        out_type=jax.ShapeDtypeStruct((S, H), jnp.float32),
        scratch_types=[
            pltpu.VMEM((_TOK_H,), jnp.int32),
            pltpu.VMEM((_TOK_H,), jnp.int32),
            pltpu.VMEM((_TOK_H, H), jnp.float32),
            pltpu.VMEM((_TOK_H, H), jnp.float32),
            pltpu.SemaphoreType.DMA,
            pltpu.SemaphoreType.DMA,
        ],
    )
    def _combine(ys_hbm, dest_hbm, out_hbm, i0, i1, b0, b1, sem0, sem1):
        wid = lax.axis_index("s") * 2 + lax.axis_index("c")
        for half in range(2):
            base = wid * _TOK_W + half * _TOK_H
            pltpu.sync_copy(dest_hbm.at[0, pl.ds(base, _TOK_H)], i0)
            pltpu.sync_copy(dest_hbm.at[1, pl.ds(base, _TOK_H)], i1)
            cp0 = pltpu.async_copy(ys_hbm.at[i0], b0, sem0)
            cp1 = pltpu.async_copy(ys_hbm.at[i1], b1, sem1)
            cp0.wait()
            cp1.wait()

            def body_j(j, _):
                def body_c(c, _):
                    sl = pl.ds(c * 16, 16)
                    b0[j, sl] = b0[j, sl] + b1[j, sl]
                    return 0

                lax.fori_loop(0, H // 16, body_c, 0)
                return 0

            lax.fori_loop(0, _TOK_H, body_j, 0)
            pltpu.sync_copy(b0, out_hbm.at[pl.ds(base, _TOK_H)])

    return _combine


@jax.jit
def _pipeline(hidden_states, router_w, gate_up_proj, gate_up_proj_bias,
              down_proj, down_proj_bias):
    b, s, h = hidden_states.shape
    x = hidden_states.reshape(s, h)
    xb = x.astype(jnp.bfloat16)
    dest, w16, meta = _plan(xb, router_w.astype(jnp.bfloat16))
    xs, wr = _make_dispatch()(x, dest, w16)
    ys = _ffn(meta[0, :NT + 8], xs, wr,
              gate_up_proj, gate_up_proj_bias,
              down_proj, down_proj_bias)
    out = _make_combine()(ys, dest)
    return out.reshape(b, s, h)


def kernel(hidden_states, router_w, gate_up_proj, gate_up_proj_bias, down_proj,
           down_proj_bias):
    return _pipeline(hidden_states, router_w, gate_up_proj, gate_up_proj_bias,
                     down_proj, down_proj_bias)


# speedup vs baseline: 1.2259x; 1.0158x over previous
import functools

import jax
import jax.numpy as jnp
from jax import lax
from jax.experimental import pallas as pl
from jax.experimental.pallas import tpu as pltpu
from jax.experimental.pallas import tpu_sc as plsc

S = 2048
H = 768
E = 8
INTER = 768
LIMIT = 7.0
ALPHA = 1.702

T = 256
TW = 128
NT = S * 2 // T + E
RPAD = NT * T
NC = 16
CH = S // NC
NW = 32


def _plan_body(x_ref, rw_ref, dest_ref, wts_ref, meta_ref):
    xb = x_ref[...]
    logits = jnp.dot(xb, rw_ref[...], preferred_element_type=jnp.float32)
    m = jnp.max(logits, axis=-1, keepdims=True)
    ex = jnp.exp(logits - m)
    scores = ex / jnp.sum(ex, axis=-1, keepdims=True)
    eidx = lax.broadcasted_iota(jnp.int32, (S, E), 1)
    a1 = jnp.min(jnp.where(logits == m, eidx, E), axis=-1, keepdims=True)
    o1 = jnp.where(eidx == a1, 1.0, 0.0)
    neg = jnp.float32(-jnp.inf)
    logits2 = jnp.where(eidx == a1, neg, logits)
    m2 = jnp.max(logits2, axis=-1, keepdims=True)
    a2 = jnp.min(jnp.where(logits2 == m2, eidx, E), axis=-1, keepdims=True)
    o2 = jnp.where(eidx == a2, 1.0, 0.0)

    w0v = jnp.sum(scores * o1, axis=1).reshape(1, S, 1)
    w1v = jnp.sum(scores * o2, axis=1).reshape(1, S, 1)
    wts_ref[pl.ds(0, 1)] = jnp.broadcast_to(w0v, (1, S, 128))
    wts_ref[pl.ds(1, 1)] = jnp.broadcast_to(w1v, (1, S, 128))

    r128 = lax.broadcasted_iota(jnp.int32, (CH, CH), 0)
    c128 = lax.broadcasted_iota(jnp.int32, (CH, CH), 1)
    Ls = jnp.where(r128 > c128, 1.0, 0.0)
    within1 = []
    within2 = []
    tots1 = []
    tots2 = []
    for c in range(NC):
        o1c = o1[c * CH:(c + 1) * CH]
        o2c = o2[c * CH:(c + 1) * CH]
        within1.append(jnp.dot(Ls, o1c, preferred_element_type=jnp.float32))
        within2.append(jnp.dot(Ls, o2c, preferred_element_type=jnp.float32))
        tots1.append(jnp.sum(o1c, axis=0, keepdims=True))
        tots2.append(jnp.sum(o2c, axis=0, keepdims=True))
    tots1 = jnp.concatenate(tots1, axis=0)
    tots2 = jnp.concatenate(tots2, axis=0)
    r16 = lax.broadcasted_iota(jnp.int32, (NC, NC), 0)
    c16 = lax.broadcasted_iota(jnp.int32, (NC, NC), 1)
    L16 = jnp.where(r16 > c16, 1.0, 0.0)
    base1 = jnp.dot(L16, tots1, preferred_element_type=jnp.float32)
    base2 = jnp.dot(L16, tots2, preferred_element_type=jnp.float32)

    c1tot = jnp.sum(tots1, axis=0, keepdims=True)
    c2tot = jnp.sum(tots2, axis=0, keepdims=True)
    counts = c1tot + c2tot
    ntiles = jnp.floor((counts + (T - 1)) * (1.0 / T))
    r8 = lax.broadcasted_iota(jnp.int32, (E, E), 0)
    c8 = lax.broadcasted_iota(jnp.int32, (E, E), 1)
    U8 = jnp.where(r8 < c8, 1.0, 0.0)
    tile_off = jnp.dot(ntiles, U8, preferred_element_type=jnp.float32)
    seg_base = tile_off * float(T)

    for c in range(NC):
        o1c = o1[c * CH:(c + 1) * CH]
        o2c = o2[c * CH:(c + 1) * CH]
        cnt1 = within1[c] + base1[c:c + 1, :]
        cnt2 = within2[c] + base2[c:c + 1, :]
        d0 = jnp.sum(o1c * (seg_base + cnt1), axis=1)
        d1 = jnp.sum(o2c * (seg_base + c1tot + cnt2), axis=1)
        dest_ref[pl.ds(0, 1), pl.ds(c * CH, CH)] = (
            d0.astype(jnp.int32).reshape(1, CH))
        dest_ref[pl.ds(1, 1), pl.ds(c * CH, CH)] = (
            d1.astype(jnp.int32).reshape(1, CH))

    i40 = lax.broadcasted_iota(jnp.int32, (NT, E), 0).astype(jnp.float32)
    eids = jnp.sum(jnp.where(tile_off <= i40, 1.0, 0.0), axis=1) - 1.0
    meta_ref[pl.ds(0, 1), pl.ds(0, NT)] = eids.astype(jnp.int32).reshape(1, NT)
    total = jnp.sum(ntiles).astype(jnp.int32)
    meta_ref[pl.ds(0, 1), pl.ds(NT, 1)] = total.reshape(1, 1)


@jax.jit
def _plan(xb, rw):
    return pl.pallas_call(
        _plan_body,
        grid=(1,),
        in_specs=[
            pl.BlockSpec((S, H), lambda i: (0, 0)),
            pl.BlockSpec((H, E), lambda i: (0, 0)),
        ],
        out_specs=[
            pl.BlockSpec((8, S), lambda i: (0, 0)),
            pl.BlockSpec((2, S, 128), lambda i: (0, 0, 0)),
            pl.BlockSpec((8, 64), lambda i: (0, 0)),
        ],
        out_shape=[
            jax.ShapeDtypeStruct((8, S), jnp.int32),
            jax.ShapeDtypeStruct((2, S, 128), jnp.float32),
            jax.ShapeDtypeStruct((8, 64), jnp.int32),
        ],
        compiler_params=pltpu.CompilerParams(
            dimension_semantics=("arbitrary",),
        ),
    )(xb, rw)


@functools.cache
def _make_dispatch():
    mesh = plsc.VectorSubcoreMesh(core_axis_name="c", subcore_axis_name="s")

    @jax.jit
    @functools.partial(
        pl.kernel,
        mesh=mesh,
        out_type=[
            jax.ShapeDtypeStruct((RPAD, H), jnp.float32),
            jax.ShapeDtypeStruct((RPAD, 128), jnp.float32),
        ],
        scratch_types=[
            pltpu.VMEM((TW,), jnp.int32),
            pltpu.VMEM((2, TW // 2), jnp.int32),
            pltpu.VMEM((TW // 2, H), jnp.float32),
            pltpu.VMEM((TW // 2, H), jnp.float32),
            pltpu.VMEM((TW, 128), jnp.float32),
            pltpu.SemaphoreType.DMA,
            pltpu.SemaphoreType.DMA,
            pltpu.SemaphoreType.DMA,
            pltpu.SemaphoreType.DMA,
        ],
    )
    def _dispatch(x_hbm, dest_hbm, w16_hbm, xs_hbm, wr_hbm,
                  idx_v, idx2_v, rows_a, rows_b, wrow_v,
                  sem_a, sem_b, sem_w, sem_i):
        wid = lax.axis_index("s") * 2 + lax.axis_index("c")
        k = wid // 16
        tbase = (wid % 16) * TW
        HT = TW // 2
        cpi = pltpu.async_copy(dest_hbm.at[k, pl.ds(tbase, TW)], idx_v, sem_i)
        cpa = pltpu.async_copy(x_hbm.at[pl.ds(tbase, HT)], rows_a, sem_a)
        cpb = pltpu.async_copy(x_hbm.at[pl.ds(tbase + HT, HT)], rows_b, sem_b)
        cpw = pltpu.async_copy(w16_hbm.at[k, pl.ds(tbase, TW)], wrow_v, sem_w)
        cpi.wait()
        pltpu.sync_copy(dest_hbm.at[k, pl.ds(tbase, HT)], idx2_v.at[0])
        pltpu.sync_copy(dest_hbm.at[k, pl.ds(tbase + HT, HT)], idx2_v.at[1])
        cpa.wait()
        sca = pltpu.async_copy(rows_a, xs_hbm.at[idx2_v.at[0]], sem_a)
        cpw.wait()
        scw = pltpu.async_copy(wrow_v, wr_hbm.at[idx_v], sem_w)
        cpb.wait()
        scb = pltpu.async_copy(rows_b, xs_hbm.at[idx2_v.at[1]], sem_b)
        sca.wait()
        scw.wait()
        scb.wait()

    return _dispatch


def _ffn_body(meta_ref, xs_ref, wr_ref, wgu_ref, bgu_ref, wd_ref, bd_ref,
              ys_ref):
    i = pl.program_id(0)

    @pl.when(i < meta_ref[NT])
    def _():
        x = xs_ref[...]
        gu = jnp.dot(x, wgu_ref[0], preferred_element_type=jnp.float32)
        gu = gu + bgu_ref[0]
        gate = jnp.minimum(gu[:, :INTER], LIMIT)
        up = jnp.clip(gu[:, INTER:], -LIMIT, LIMIT)
        glu = gate * jax.nn.sigmoid(gate * ALPHA)
        act = (up + 1.0) * glu
        y = jnp.dot(act, wd_ref[0], preferred_element_type=jnp.float32)
        ys_ref[...] = (y + bd_ref[0]) * wr_ref[...][:, 0:1]


@jax.jit
def _ffn(meta1d, xs, wr, wgu, bgu, wd, bd):
    grid_spec = pltpu.PrefetchScalarGridSpec(
        num_scalar_prefetch=1,
        grid=(NT,),
        in_specs=[
            pl.BlockSpec((T, H), lambda i, m: (i, 0)),
            pl.BlockSpec((T, 128), lambda i, m: (i, 0)),
            pl.BlockSpec((1, H, 2 * INTER), lambda i, m: (m[i], 0, 0)),
            pl.BlockSpec((1, 1, 2 * INTER), lambda i, m: (m[i], 0, 0)),
            pl.BlockSpec((1, H, INTER), lambda i, m: (m[i], 0, 0)),
            pl.BlockSpec((1, 1, H), lambda i, m: (m[i], 0, 0)),
        ],
        out_specs=pl.BlockSpec((T, H), lambda i, m: (i, 0)),
    )
    return pl.pallas_call(
        _ffn_body,
        grid_spec=grid_spec,
        out_shape=jax.ShapeDtypeStruct((RPAD, H), jnp.float32),
        compiler_params=pltpu.CompilerParams(
            dimension_semantics=("parallel",),
        ),
    )(meta1d, xs, wr, wgu, bgu.reshape(E, 1, 2 * INTER), wd,
      bd.reshape(E, 1, H))


_TOK_W = S // NW
_TOK_H = _TOK_W


@functools.cache
def _make_combine():
    mesh = plsc.VectorSubcoreMesh(core_axis_name="c", subcore_axis_name="s")

    @jax.jit
    @functools.partial(
        pl.kernel,
        mesh=mesh,
        out_type=jax.ShapeDtypeStruct((S, H), jnp.float32),
        scratch_types=[
            pltpu.VMEM((_TOK_H,), jnp.int32),
            pltpu.VMEM((_TOK_H,), jnp.int32),
            pltpu.VMEM((_TOK_H, H), jnp.float32),
            pltpu.VMEM((_TOK_H, H), jnp.float32),
            pltpu.SemaphoreType.DMA,
            pltpu.SemaphoreType.DMA,
        ],
    )
    def _combine(ys_hbm, dest_hbm, out_hbm, i0, i1, b0, b1, sem0, sem1):
        wid = lax.axis_index("s") * 2 + lax.axis_index("c")
        for half in range(1):
            base = wid * _TOK_W + half * _TOK_H
            pltpu.sync_copy(dest_hbm.at[0, pl.ds(base, _TOK_H)], i0)
            pltpu.sync_copy(dest_hbm.at[1, pl.ds(base, _TOK_H)], i1)
            cp0 = pltpu.async_copy(ys_hbm.at[i0], b0, sem0)
            cp1 = pltpu.async_copy(ys_hbm.at[i1], b1, sem1)
            cp0.wait()
            cp1.wait()

            def body_j(j, _):
                def body_c(c, _):
                    sl = pl.ds(c * 16, 16)
                    b0[j, sl] = b0[j, sl] + b1[j, sl]
                    return 0

                lax.fori_loop(0, H // 16, body_c, 0)
                return 0

            lax.fori_loop(0, _TOK_H, body_j, 0)
            pltpu.sync_copy(b0, out_hbm.at[pl.ds(base, _TOK_H)])

    return _combine


@jax.jit
def _pipeline(hidden_states, router_w, gate_up_proj, gate_up_proj_bias,
              down_proj, down_proj_bias):
    b, s, h = hidden_states.shape
    x = hidden_states.reshape(s, h)
    xb = x.astype(jnp.bfloat16)
    dest, w16, meta = _plan(xb, router_w.astype(jnp.bfloat16))
    xs, wr = _make_dispatch()(x, dest, w16)
    ys = _ffn(meta[0, :NT + 8], xs, wr,
              gate_up_proj, gate_up_proj_bias,
              down_proj, down_proj_bias)
    out = _make_combine()(ys, dest)
    return out.reshape(b, s, h)


def kernel(hidden_states, router_w, gate_up_proj, gate_up_proj_bias, down_proj,
           down_proj_bias):
    return _pipeline(hidden_states, router_w, gate_up_proj, gate_up_proj_bias,
                     down_proj, down_proj_bias)
